# baseline (speedup 1.0000x reference)
import functools
import math

import jax
import jax.numpy as jnp
from jax import lax
from jax.experimental import pallas as pl
from jax.experimental.pallas import tpu as pltpu
from jax.experimental.pallas import tpu_sc as plsc

D_MODEL = 64
SCALE = math.sqrt(D_MODEL)

NC = 2
NS = 16
NW = NC * NS

SUB = 128
CHUNK_ROWS = 512
NSUB = CHUNK_ROWS // SUB


@functools.partial(jax.jit, static_argnames=("B",))
def _embed(idx, table, B):
    b_per_w = B // NW
    n_chunks = b_per_w // CHUNK_ROWS
    irows_per_w = b_per_w // SUB
    mesh = plsc.VectorSubcoreMesh(core_axis_name="c", subcore_axis_name="s")

    @functools.partial(
        pl.kernel,
        mesh=mesh,
        out_type=jax.ShapeDtypeStruct((B, D_MODEL), jnp.float32),
        scratch_types=[
            pltpu.VMEM((NSUB, SUB), jnp.int32),
            pltpu.VMEM((CHUNK_ROWS, D_MODEL), jnp.float32),
            pltpu.SemaphoreType.DMA,
        ],
        compiler_params=pltpu.CompilerParams(use_tc_tiling_on_sc=False),
    )
    def k(idx_hbm, table_hbm, out_hbm, idx_v, rows_v, sem):
        wid = lax.axis_index("s") * NC + lax.axis_index("c")
        irow0 = wid * irows_per_w
        out0 = wid * b_per_w

        def chunk_body(ci, carry):
            pltpu.sync_copy(idx_hbm.at[pl.ds(irow0 + ci * NSUB, NSUB)], idx_v)
            copies = [
                pltpu.async_copy(
                    table_hbm.at[idx_v.at[j]],
                    rows_v.at[pl.ds(j * SUB, SUB)],
                    sem,
                )
                for j in range(NSUB)
            ]
            for c in copies:
                c.wait()

            def mul_body(r, carry2):
                for s in range(D_MODEL // 16):
                    rows_v[r, pl.ds(s * 16, 16)] = (
                        rows_v[r, pl.ds(s * 16, 16)] * SCALE
                    )
                return carry2

            lax.fori_loop(0, CHUNK_ROWS, mul_body, 0)
            pltpu.sync_copy(
                rows_v, out_hbm.at[pl.ds(out0 + ci * CHUNK_ROWS, CHUNK_ROWS)]
            )
            return carry

        lax.fori_loop(0, n_chunks, chunk_body, 0)

    return k(idx, table)


def kernel(x, table):
    B = x.shape[0] * x.shape[1]
    idx = x.reshape(B // SUB, SUB).astype(jnp.int32)
    out = _embed(idx, table, B)
    return out.reshape(x.shape[0], x.shape[1], D_MODEL)


# speedup vs baseline: 1.1238x; 1.1238x over previous
import functools
import math

import jax
import jax.numpy as jnp
from jax import lax
from jax.experimental import pallas as pl
from jax.experimental.pallas import tpu as pltpu
from jax.experimental.pallas import tpu_sc as plsc

D_MODEL = 64
SCALE = math.sqrt(D_MODEL)

NC = 2
NS = 16
NW = NC * NS

SUB = 128
CHUNK_ROWS = 512
NSUB = CHUNK_ROWS // SUB


@functools.partial(jax.jit, static_argnames=("B",))
def _embed(idx, table, B):
    b_per_w = B // NW
    n_chunks = b_per_w // CHUNK_ROWS
    irows_per_w = b_per_w // SUB
    assert n_chunks % 2 == 0
    mesh = plsc.VectorSubcoreMesh(core_axis_name="c", subcore_axis_name="s")

    @functools.partial(
        pl.kernel,
        mesh=mesh,
        out_type=jax.ShapeDtypeStruct((B, D_MODEL), jnp.float32),
        scratch_types=[
            pltpu.VMEM((2, NSUB, SUB), jnp.int32),
            pltpu.VMEM((2, CHUNK_ROWS, D_MODEL), jnp.float32),
            pltpu.SemaphoreType.DMA,
            pltpu.SemaphoreType.DMA,
        ],
        compiler_params=pltpu.CompilerParams(use_tc_tiling_on_sc=False),
    )
    def k(idx_hbm, table_hbm, out_hbm, idx_v, rows_v, sem0, sem1):
        wid = lax.axis_index("s") * NC + lax.axis_index("c")
        irow0 = wid * irows_per_w
        out0 = wid * b_per_w
        sems = (sem0, sem1)

        def fetch(ci, b):
            pltpu.sync_copy(
                idx_hbm.at[pl.ds(irow0 + ci * NSUB, NSUB)], idx_v.at[b]
            )
            for j in range(NSUB):
                pltpu.async_copy(
                    table_hbm.at[idx_v.at[b, j]],
                    rows_v.at[b, pl.ds(j * SUB, SUB)],
                    sems[b],
                )

        def drain(b):
            pltpu.make_async_copy(
                out_hbm.at[pl.ds(0, CHUNK_ROWS)], rows_v.at[b], sems[b]
            ).wait()

        fetch(0, 0)

        def outer(g, carry):
            for b in range(2):
                ci = g * 2 + b
                nxt = ci + 1

                @pl.when(nxt < n_chunks)
                def _():
                    fetch(nxt, 1 - b)

                drain(b)

                @functools.partial(
                    plsc.parallel_loop, 0, CHUNK_ROWS, unroll=8
                )
                def _(r):
                    for s in range(D_MODEL // 16):
                        rows_v[b, r, pl.ds(s * 16, 16)] = (
                            rows_v[b, r, pl.ds(s * 16, 16)] * SCALE
                        )

                pltpu.sync_copy(
                    rows_v.at[b],
                    out_hbm.at[pl.ds(out0 + ci * CHUNK_ROWS, CHUNK_ROWS)],
                )
            return carry

        lax.fori_loop(0, n_chunks // 2, outer, 0)

    return k(idx, table)


def kernel(x, table):
    B = x.shape[0] * x.shape[1]
    idx = x.reshape(B // SUB, SUB).astype(jnp.int32)
    out = _embed(idx, table, B)
    return out.reshape(x.shape[0], x.shape[1], D_MODEL)
